# baseline (speedup 1.0000x reference)
import jax
import jax.numpy as jnp
from jax import lax
from jax.experimental import pallas as pl
from jax.experimental.pallas import tpu as pltpu
from jax.experimental.pallas import tpu_sc as plsc

NU = 25000
NI = 25000
N = NU + NI
D = 64
E = 800000
N_LAYERS = 3

NC = 2
NS = 16
NW = NC * NS
PASSES = D // NW

BLK = 128
CH_ROWS = 50
E_PAD = 819200
NROWS = E_PAD // BLK
NCHUNKS = NROWS // CH_ROWS

WSCALE = 65535.0


def _prop_body(tableT, sd2d, w2d, out1, out2, out3,
               bufa, bufb, sd0, sd1, w0, w1, sem0, sem1):
    c = lax.axis_index("c")
    s = lax.axis_index("s")
    wid = s * NC + c
    sdbufs = (sd0, sd1)
    wbufs = (w0, w1)
    sems = (sem0, sem1)
    zero16 = jnp.zeros((16,), jnp.float32)
    sh16 = jnp.full((16,), 16, jnp.int32)
    m16 = jnp.full((16,), 0xFFFF, jnp.int32)
    outs = (out1, out2, out3)

    inv_ws = jnp.float32(1.0 / WSCALE)

    for p in range(PASSES):
        col = wid + NW * p
        pltpu.sync_copy(tableT.at[col], bufa)
        gbuf, abuf = bufa, bufb

        for layer in range(N_LAYERS):
            @pl.loop(0, N // 16)
            def _(i):
                gbuf[pl.ds(i * 16, 16)] = gbuf[pl.ds(i * 16, 16)] * inv_ws
                abuf[pl.ds(i * 16, 16)] = zero16

            pltpu.async_copy(sd2d.at[pl.ds(0, CH_ROWS * 4)], sd0, sem0)
            pltpu.async_copy(w2d.at[pl.ds(0, CH_ROWS * 4)], w0, sem0)

            @pl.loop(0, NCHUNKS, step=2)
            def _(k2):
                for par in range(2):
                    k = k2 + par
                    sdb, wb, sm = sdbufs[par], wbufs[par], sems[par]
                    nsdb, nwb, nsm = (sdbufs[1 - par], wbufs[1 - par],
                                      sems[1 - par])

                    @pl.when(k + 1 < NCHUNKS)
                    def _():
                        row0 = (k + 1) * CH_ROWS * 4
                        pltpu.async_copy(
                            sd2d.at[pl.ds(row0, CH_ROWS * 4)], nsdb, nsm)
                        pltpu.async_copy(
                            w2d.at[pl.ds(row0, CH_ROWS * 4)], nwb, nsm)

                    pltpu.make_async_copy(
                        sd2d.at[pl.ds(0, CH_ROWS * 4)], sdb, sm).wait()
                    pltpu.make_async_copy(
                        w2d.at[pl.ds(0, CH_ROWS * 4)], wb, sm).wait()

                    @plsc.parallel_loop(0, CH_ROWS * (BLK // 32))
                    def _(r):
                        wpkv = wb[r, :]
                        wlo = lax.convert_element_type(wpkv & m16, jnp.float32)
                        whi = lax.convert_element_type(
                            lax.shift_right_logical(wpkv, sh16), jnp.float32)
                        for v, wv in ((0, wlo), (1, whi)):
                            sdv = sdb[r, pl.ds(v * 16, 16)]
                            srcv = sdv & m16
                            dstv = lax.shift_right_logical(sdv, sh16)
                            g = plsc.load_gather(gbuf, [srcv])
                            plsc.addupdate_scatter(abuf, [dstv], g * wv)

            pltpu.sync_copy(abuf, outs[layer].at[col])
            gbuf, abuf = abuf, gbuf


_SDS = jax.ShapeDtypeStruct((D, N), jnp.float32)
_prop = pl.kernel(
    _prop_body,
    out_type=(_SDS, _SDS, _SDS),
    mesh=plsc.VectorSubcoreMesh(core_axis_name="c", subcore_axis_name="s"),
    compiler_params=pltpu.CompilerParams(use_tc_tiling_on_sc=False,
                                         needs_layout_passes=False),
    scratch_types=[
        pltpu.VMEM((N,), jnp.float32),
        pltpu.VMEM((N,), jnp.float32),
        pltpu.VMEM((CH_ROWS * (BLK // 32), 32), jnp.int32),
        pltpu.VMEM((CH_ROWS * (BLK // 32), 32), jnp.int32),
        pltpu.VMEM((CH_ROWS * (BLK // 32), 16), jnp.int32),
        pltpu.VMEM((CH_ROWS * (BLK // 32), 16), jnp.int32),
        pltpu.SemaphoreType.DMA,
        pltpu.SemaphoreType.DMA,
    ],
)


def _mean_body(a, b, c, d, o):
    o[...] = (a[...] + b[...] + c[...] + d[...]) * 0.25


def _mean4(e0, e1, e2, e3):
    spec = pl.BlockSpec((D // 4, N), lambda i: (i, 0))
    return pl.pallas_call(
        _mean_body,
        grid=(4,),
        in_specs=[spec] * 4,
        out_specs=spec,
        out_shape=jax.ShapeDtypeStruct((D, N), jnp.float32),
    )(e0, e1, e2, e3)


def kernel(user_emb, item_emb, edge_index, edge_weight):
    embT0 = jnp.concatenate([user_emb, item_emb], axis=0).T

    pad = E_PAD - E
    src = jnp.concatenate([edge_index[0], jnp.zeros((pad,), jnp.int32)])
    dst = jnp.concatenate([edge_index[1], jnp.zeros((pad,), jnp.int32)])
    sd = (src | (dst << 16)).reshape(NROWS * (BLK // 32), 32)
    wq = jnp.round(
        jnp.concatenate([edge_weight, jnp.zeros((pad,), jnp.float32)])
        * WSCALE).astype(jnp.int32).reshape(-1, 2, 16)
    w = (wq[:, 0, :] | (wq[:, 1, :] << 16)).reshape(NROWS * (BLK // 32), 16)

    e1, e2, e3 = _prop(embT0, sd, w)
    light_out = _mean4(embT0, e1, e2, e3).T
    return light_out[:NU], light_out[NU:]


# speedup vs baseline: 1.0163x; 1.0163x over previous
import jax
import jax.numpy as jnp
from jax import lax
from jax.experimental import pallas as pl
from jax.experimental.pallas import tpu as pltpu
from jax.experimental.pallas import tpu_sc as plsc

NU = 25000
NI = 25000
N = NU + NI
D = 64
E = 800000
N_LAYERS = 3

NC = 2
NS = 16
NW = NC * NS
PASSES = D // NW

BLK = 128
CH_ROWS = 50
E_PAD = 819200
NROWS = E_PAD // BLK
NCHUNKS = NROWS // CH_ROWS


def _prop_body(tableT, sd2d, w2d, out1, out2, out3,
               bufa, bufb, sd0, sd1, w0, w1, sem0, sem1):
    c = lax.axis_index("c")
    s = lax.axis_index("s")
    wid = s * NC + c
    sdbufs = (sd0, sd1)
    wbufs = (w0, w1)
    sems = (sem0, sem1)
    zero16 = jnp.zeros((16,), jnp.float32)
    sh16 = jnp.full((16,), 16, jnp.int32)
    m16 = jnp.full((16,), 0xFFFF, jnp.int32)
    outs = (out1, out2, out3)

    for p in range(PASSES):
        col = wid + NW * p
        pltpu.sync_copy(tableT.at[col], bufa)
        gbuf, abuf = bufa, bufb

        for layer in range(N_LAYERS):
            @pl.loop(0, N // 16)
            def _(i):
                abuf[pl.ds(i * 16, 16)] = zero16

            pltpu.async_copy(sd2d.at[pl.ds(0, CH_ROWS * 4)], sd0, sem0)
            pltpu.async_copy(w2d.at[pl.ds(0, CH_ROWS * 4)], w0, sem0)

            @pl.loop(0, NCHUNKS, step=2)
            def _(k2):
                for par in range(2):
                    k = k2 + par
                    sdb, wb, sm = sdbufs[par], wbufs[par], sems[par]
                    nsdb, nwb, nsm = (sdbufs[1 - par], wbufs[1 - par],
                                      sems[1 - par])

                    @pl.when(k + 1 < NCHUNKS)
                    def _():
                        row0 = (k + 1) * CH_ROWS * 4
                        pltpu.async_copy(
                            sd2d.at[pl.ds(row0, CH_ROWS * 4)], nsdb, nsm)
                        pltpu.async_copy(
                            w2d.at[pl.ds(row0, CH_ROWS * 4)], nwb, nsm)

                    pltpu.make_async_copy(
                        sd2d.at[pl.ds(0, CH_ROWS * 4)], sdb, sm).wait()
                    pltpu.make_async_copy(
                        w2d.at[pl.ds(0, CH_ROWS * 4)], wb, sm).wait()

                    @plsc.parallel_loop(0, CH_ROWS * (BLK // 32))
                    def _(r):
                        for v in range(2):
                            sdv = sdb[r, pl.ds(v * 16, 16)]
                            wv = wb[r, pl.ds(v * 16, 16)]
                            srcv = sdv & m16
                            dstv = lax.shift_right_logical(sdv, sh16)
                            g = plsc.load_gather(gbuf, [srcv])
                            plsc.addupdate_scatter(abuf, [dstv], g * wv)

            pltpu.sync_copy(abuf, outs[layer].at[col])
            gbuf, abuf = abuf, gbuf


_SDS = jax.ShapeDtypeStruct((D, N), jnp.float32)
_prop = pl.kernel(
    _prop_body,
    out_type=(_SDS, _SDS, _SDS),
    mesh=plsc.VectorSubcoreMesh(core_axis_name="c", subcore_axis_name="s"),
    compiler_params=pltpu.CompilerParams(use_tc_tiling_on_sc=False,
                                         needs_layout_passes=False),
    scratch_types=[
        pltpu.VMEM((N,), jnp.float32),
        pltpu.VMEM((N,), jnp.float32),
        pltpu.VMEM((CH_ROWS * (BLK // 32), 32), jnp.int32),
        pltpu.VMEM((CH_ROWS * (BLK // 32), 32), jnp.int32),
        pltpu.VMEM((CH_ROWS * (BLK // 32), 32), jnp.float32),
        pltpu.VMEM((CH_ROWS * (BLK // 32), 32), jnp.float32),
        pltpu.SemaphoreType.DMA,
        pltpu.SemaphoreType.DMA,
    ],
)


def _mean_body(a, b, c, d, o):
    o[...] = (a[...] + b[...] + c[...] + d[...]) * 0.25


def _mean4(e0, e1, e2, e3):
    spec = pl.BlockSpec((D // 4, N), lambda i: (i, 0))
    return pl.pallas_call(
        _mean_body,
        grid=(4,),
        in_specs=[spec] * 4,
        out_specs=spec,
        out_shape=jax.ShapeDtypeStruct((D, N), jnp.float32),
    )(e0, e1, e2, e3)


def kernel(user_emb, item_emb, edge_index, edge_weight):
    embT0 = jnp.concatenate([user_emb, item_emb], axis=0).T

    pad = E_PAD - E
    src = jnp.concatenate([edge_index[0], jnp.zeros((pad,), jnp.int32)])
    dst = jnp.concatenate([edge_index[1], jnp.zeros((pad,), jnp.int32)])
    sd = (src | (dst << 16)).reshape(NROWS * (BLK // 32), 32)
    w = jnp.concatenate(
        [edge_weight, jnp.zeros((pad,), jnp.float32)]).reshape(
            NROWS * (BLK // 32), 32)

    e1, e2, e3 = _prop(embT0, sd, w)
    light_out = _mean4(embT0, e1, e2, e3).T
    return light_out[:NU], light_out[NU:]
